# Initial kernel scaffold; baseline (speedup 1.0000x reference)
import functools

import jax
import jax.numpy as jnp
from jax import lax
from jax.experimental import pallas as pl
from jax.experimental.pallas import tpu as pltpu
from jax.experimental.pallas import tpu_sc as plsc

NC = 2
NS = 16
NW = NC * NS

VOCAB = 100000
EMBED = 64
SEQ = 50
BATCH = 1024
SAMPLE = 20


def _sc_gather(table, idx, W, K):
    N = idx.shape[0]
    D = table.shape[1]
    n_per_w = N // NW
    R = n_per_w // W
    n_chunks = R // K
    assert N == NW * R * W and R == n_chunks * K

    mesh = plsc.VectorSubcoreMesh(core_axis_name="c", subcore_axis_name="s")

    @functools.partial(
        pl.kernel,
        mesh=mesh,
        out_type=jax.ShapeDtypeStruct((N, D), jnp.float32),
        scratch_types=[
            pltpu.VMEM((R, W), jnp.int32),
            pltpu.VMEM((K * W, D), jnp.float32),
            pltpu.SemaphoreType.DMA,
        ],
    )
    def gather_kernel(table_hbm, idx_hbm, out_hbm, idx_v, rows_v, sem):
        wid = lax.axis_index("s") * NC + lax.axis_index("c")
        base = wid * n_per_w
        pltpu.sync_copy(
            idx_hbm.at[pl.ds(base, n_per_w)],
            idx_v.reshape(n_per_w),
        )

        def chunk_body(i, carry):
            cps = []
            for j in range(K):
                cps.append(
                    pltpu.async_copy(
                        table_hbm.at[idx_v.at[i * K + j]],
                        rows_v.at[pl.ds(j * W, W)],
                        sem,
                    )
                )
            for cp in cps:
                cp.wait()
            pltpu.sync_copy(rows_v, out_hbm.at[pl.ds(base + i * (K * W), K * W)])
            return carry

        lax.fori_loop(0, n_chunks, chunk_body, 0)

    return gather_kernel(table, idx)


def _lstm_body(x_ref, wih_ref, whh_ref, b_ref, out_ref, h_scr, c_scr):
    t = pl.program_id(0)

    @pl.when(t == 0)
    def _init():
        h_scr[...] = jnp.zeros_like(h_scr)
        c_scr[...] = jnp.zeros_like(c_scr)

    xt = x_ref[0]
    gates = (
        jnp.dot(xt, wih_ref[...], preferred_element_type=jnp.float32)
        + jnp.dot(h_scr[...], whh_ref[...], preferred_element_type=jnp.float32)
        + b_ref[...]
    )
    i = jax.nn.sigmoid(gates[:, 0 * EMBED : 1 * EMBED])
    f = jax.nn.sigmoid(gates[:, 1 * EMBED : 2 * EMBED])
    g = jnp.tanh(gates[:, 2 * EMBED : 3 * EMBED])
    o = jax.nn.sigmoid(gates[:, 3 * EMBED : 4 * EMBED])
    c = f * c_scr[...] + i * g
    h = o * jnp.tanh(c)
    c_scr[...] = c
    h_scr[...] = h
    out_ref[0] = h


def _lstm(x, wih_t, whh_t, b):
    T, B, E = x.shape
    G = 4 * E
    return pl.pallas_call(
        _lstm_body,
        grid=(T,),
        in_specs=[
            pl.BlockSpec((1, B, E), lambda t: (t, 0, 0)),
            pl.BlockSpec((E, G), lambda t: (0, 0)),
            pl.BlockSpec((E, G), lambda t: (0, 0)),
            pl.BlockSpec((1, G), lambda t: (0, 0)),
        ],
        out_specs=pl.BlockSpec((1, B, E), lambda t: (t, 0, 0)),
        out_shape=jax.ShapeDtypeStruct((T, B, E), jnp.float32),
        scratch_shapes=[
            pltpu.VMEM((B, E), jnp.float32),
            pltpu.VMEM((B, E), jnp.float32),
        ],
    )(x, wih_t, whh_t, b)


def kernel(samples, text, targets, in_embed, out_embed, W_ih, W_hh, b_ih, b_hh):
    E = in_embed.shape[1]
    sample_size = samples.shape[-1]

    txt_idx = text.reshape(-1).astype(jnp.int32)
    tgt_idx = targets.reshape(-1).astype(jnp.int32)
    samp_idx = samples.reshape(-1).astype(jnp.int32)

    txt_emb = _sc_gather(in_embed, txt_idx, W=64, K=5)
    rnn = _lstm(
        txt_emb.reshape(SEQ, BATCH, E),
        W_ih.T,
        W_hh.T,
        (b_ih + b_hh).reshape(1, -1),
    )
    tgt_emb = _sc_gather(out_embed, tgt_idx, W=64, K=5)
    samp_emb = _sc_gather(out_embed, samp_idx, W=128, K=10)

    return (
        samp_emb.reshape(-1, sample_size, E),
        rnn.reshape(-1, E)[:, :, None],
        tgt_emb[:, None, :],
    )


# speedup vs baseline: 4.8335x; 4.8335x over previous
import functools

import jax
import jax.numpy as jnp
from jax import lax
from jax.experimental import pallas as pl
from jax.experimental.pallas import tpu as pltpu
from jax.experimental.pallas import tpu_sc as plsc

NC = 2
NS = 16
NW = NC * NS

VOCAB = 100000
EMBED = 64
SEQ = 50
BATCH = 1024
SAMPLE = 20


def _sc_gather(table, idx, W, K):
    N = idx.shape[0]
    D = table.shape[1]
    n_per_w = N // NW
    R = n_per_w // W
    n_chunks = R // K
    assert N == NW * R * W and R == n_chunks * K

    mesh = plsc.VectorSubcoreMesh(core_axis_name="c", subcore_axis_name="s")

    @functools.partial(
        pl.kernel,
        mesh=mesh,
        out_type=jax.ShapeDtypeStruct((N, D), jnp.float32),
        compiler_params=pltpu.CompilerParams(use_tc_tiling_on_sc=False),
        scratch_types=[
            pltpu.VMEM((R, W), jnp.int32),
            pltpu.VMEM((K * W, D), jnp.float32),
            pltpu.SemaphoreType.DMA,
        ],
    )
    def gather_kernel(table_hbm, idx_hbm, out_hbm, idx_v, rows_v, sem):
        wid = lax.axis_index("s") * NC + lax.axis_index("c")
        base = wid * n_per_w
        pltpu.sync_copy(idx_hbm.at[wid], idx_v)

        def chunk_body(i, carry):
            cps = []
            for j in range(K):
                cps.append(
                    pltpu.async_copy(
                        table_hbm.at[idx_v.at[i * K + j]],
                        rows_v.at[pl.ds(j * W, W)],
                        sem,
                    )
                )
            for cp in cps:
                cp.wait()
            pltpu.sync_copy(rows_v, out_hbm.at[pl.ds(base + i * (K * W), K * W)])
            return carry

        lax.fori_loop(0, n_chunks, chunk_body, 0)

    return gather_kernel(table, idx.reshape(NW, R, W))


def _lstm_body(x_ref, wih_ref, whh_ref, b_ref, out_ref, h_scr, c_scr):
    t = pl.program_id(0)

    @pl.when(t == 0)
    def _init():
        h_scr[...] = jnp.zeros_like(h_scr)
        c_scr[...] = jnp.zeros_like(c_scr)

    xt = x_ref[0]
    gates = (
        jnp.dot(xt, wih_ref[...], preferred_element_type=jnp.float32)
        + jnp.dot(h_scr[...], whh_ref[...], preferred_element_type=jnp.float32)
        + b_ref[...]
    )
    i = jax.nn.sigmoid(gates[:, 0 * EMBED : 1 * EMBED])
    f = jax.nn.sigmoid(gates[:, 1 * EMBED : 2 * EMBED])
    g = jnp.tanh(gates[:, 2 * EMBED : 3 * EMBED])
    o = jax.nn.sigmoid(gates[:, 3 * EMBED : 4 * EMBED])
    c = f * c_scr[...] + i * g
    h = o * jnp.tanh(c)
    c_scr[...] = c
    h_scr[...] = h
    out_ref[0] = h


def _lstm(x, wih_t, whh_t, b):
    T, B, E = x.shape
    G = 4 * E
    return pl.pallas_call(
        _lstm_body,
        grid=(T,),
        in_specs=[
            pl.BlockSpec((1, B, E), lambda t: (t, 0, 0)),
            pl.BlockSpec((E, G), lambda t: (0, 0)),
            pl.BlockSpec((E, G), lambda t: (0, 0)),
            pl.BlockSpec((1, G), lambda t: (0, 0)),
        ],
        out_specs=pl.BlockSpec((1, B, E), lambda t: (t, 0, 0)),
        out_shape=jax.ShapeDtypeStruct((T, B, E), jnp.float32),
        scratch_shapes=[
            pltpu.VMEM((B, E), jnp.float32),
            pltpu.VMEM((B, E), jnp.float32),
        ],
    )(x, wih_t, whh_t, b)


def kernel(samples, text, targets, in_embed, out_embed, W_ih, W_hh, b_ih, b_hh):
    E = in_embed.shape[1]
    sample_size = samples.shape[-1]

    txt_idx = text.reshape(-1).astype(jnp.int32)
    tgt_idx = targets.reshape(-1).astype(jnp.int32)
    samp_idx = samples.reshape(-1).astype(jnp.int32)

    txt_emb = _sc_gather(in_embed, txt_idx, W=64, K=5)
    rnn = _lstm(
        txt_emb.reshape(SEQ, BATCH, E),
        W_ih.T,
        W_hh.T,
        (b_ih + b_hh).reshape(1, -1),
    )
    tgt_emb = _sc_gather(out_embed, tgt_idx, W=64, K=5)
    samp_emb = _sc_gather(out_embed, samp_idx, W=128, K=10)

    return (
        samp_emb.reshape(-1, sample_size, E),
        rnn.reshape(-1, E)[:, :, None],
        tgt_emb[:, None, :],
    )
